# baseline (speedup 1.0000x reference)
import jax
import jax.numpy as jnp
from jax.experimental import pallas as pl
from jax.experimental.pallas import tpu as pltpu


def _combine_body(t_ref, ac_ref, om_ref, x_ref, n_ref, o_ref):
    b = pl.program_id(0)
    tt = t_ref[b]
    c1 = ac_ref[tt]
    c2 = om_ref[tt]
    o_ref[...] = c1 * x_ref[...] + c2 * n_ref[...]


def kernel(x_start, t, noise, sqrt_alphas_cumprod, sqrt_one_minus_alphas_cumprod):
    B, C, H, W = x_start.shape
    HB = 128

    smem = pl.BlockSpec(memory_space=pltpu.SMEM)
    blk = pl.BlockSpec((1, C, HB, W), lambda b, h: (b, 0, h, 0))

    out = pl.pallas_call(
        _combine_body,
        grid=(B, H // HB),
        in_specs=[smem, smem, smem, blk, blk],
        out_specs=blk,
        out_shape=jax.ShapeDtypeStruct((B, C, H, W), jnp.float32),
    )(t.astype(jnp.int32), sqrt_alphas_cumprod, sqrt_one_minus_alphas_cumprod,
      x_start, noise)
    return out


# speedup vs baseline: 1.4282x; 1.4282x over previous
import jax
import jax.numpy as jnp
from jax.experimental import pallas as pl
from jax.experimental.pallas import tpu as pltpu


def _combine_body(t_ref, ac_ref, om_ref, x_ref, n_ref, o_ref):
    b = pl.program_id(0)
    tt = t_ref[b]
    c1 = ac_ref[tt]
    c2 = om_ref[tt]
    o_ref[...] = c1 * x_ref[...] + c2 * n_ref[...]


def kernel(x_start, t, noise, sqrt_alphas_cumprod, sqrt_one_minus_alphas_cumprod):
    B, C, H, W = x_start.shape
    HB = 512

    smem = pl.BlockSpec(memory_space=pltpu.SMEM)
    blk = pl.BlockSpec((1, C, HB, W), lambda b, h: (b, 0, h, 0))

    out = pl.pallas_call(
        _combine_body,
        grid=(B, H // HB),
        in_specs=[smem, smem, smem, blk, blk],
        out_specs=blk,
        out_shape=jax.ShapeDtypeStruct((B, C, H, W), jnp.float32),
    )(t.astype(jnp.int32), sqrt_alphas_cumprod, sqrt_one_minus_alphas_cumprod,
      x_start, noise)
    return out
